# baseline (speedup 1.0000x reference)
import functools

import jax
import jax.numpy as jnp
from jax import lax
from jax.experimental import pallas as pl
from jax.experimental.pallas import tpu as pltpu
from jax.experimental.pallas import tpu_sc as plsc


def _gather_kernel(n_rows, num_cores, num_workers, seq, nbuf):
  per_w = n_rows // num_workers
  half = 128
  n_chunks = per_w // half
  groups = n_chunks // nbuf

  mesh = plsc.VectorSubcoreMesh(core_axis_name="c", subcore_axis_name="s")

  scratch = (
      [pltpu.VMEM((n_chunks, half), jnp.int32)]
      + [pltpu.VMEM((half, 128), jnp.float32) for _ in range(nbuf)]
      + [pltpu.SemaphoreType.DMA for _ in range(2 * nbuf + 1)]
  )

  @functools.partial(
      pl.kernel,
      out_type=jax.ShapeDtypeStruct((n_rows, 128), jnp.float32),
      mesh=mesh,
      scratch_types=scratch,
      compiler_params=pltpu.CompilerParams(use_tc_tiling_on_sc=True),
  )
  def run(table, idx_hbm, out, idx_v, *rest):
    bufs = rest[:nbuf]
    gsem = rest[nbuf:2 * nbuf]
    osem = rest[2 * nbuf:3 * nbuf]
    isem = rest[3 * nbuf]

    wid = lax.axis_index("s") * num_cores + lax.axis_index("c")
    base = wid * per_w

    pltpu.async_copy(idx_hbm.at[wid], idx_v, isem).wait()

    def start_gather(k, b):
      pltpu.async_copy(table.at[idx_v.at[k]], bufs[b], gsem[b])

    def wait_gather(b):
      pltpu.make_async_copy(out.at[pl.ds(base, half)], bufs[b], gsem[b]).wait()

    def start_out(k, b):
      pltpu.async_copy(bufs[b], out.at[pl.ds(base + k * half, half)], osem[b])

    def wait_out(b):
      pltpu.make_async_copy(bufs[b], out.at[pl.ds(base, half)], osem[b]).wait()

    for b in range(nbuf):
      start_gather(b, b)

    def group_body(g, carry):
      for b in range(nbuf):
        wait_gather(b)
        start_out(g * nbuf + b, b)
      for b in range(nbuf):
        wait_out(b)
        start_gather((g + 1) * nbuf + b, b)
      return carry

    lax.fori_loop(0, groups - 1, group_body, 0)

    for b in range(nbuf):
      wait_gather(b)
      start_out((groups - 1) * nbuf + b, b)
    for b in range(nbuf):
      wait_out(b)

  return run


def kernel(inlets, weight):
  b, s = inlets.shape
  v, d = weight.shape

  info = plsc.get_sparse_core_info()
  num_workers = info.num_cores * info.num_subcores

  s_sc = s // 2
  inl_sc = inlets[:, :s_sc]
  inl_tc = inlets[:, s_sc:]

  n_sc = b * s_sc
  w128 = jnp.pad(weight, ((0, 0), (0, 128 - d)))
  idx = inl_sc.astype(jnp.int32).reshape(num_workers,
                                         n_sc // (num_workers * 128), 128)
  run = _gather_kernel(n_sc, info.num_cores, num_workers, s_sc, 4)
  out_sc = run(w128, idx)[:, :d].reshape(b, s_sc, d)
  out_tc = jnp.take(weight, inl_tc, axis=0)
  return jnp.concatenate([out_sc, out_tc], axis=1)


# speedup vs baseline: 1.9249x; 1.9249x over previous
import functools

import jax
import jax.numpy as jnp
from jax import lax
from jax.experimental import pallas as pl
from jax.experimental.pallas import tpu as pltpu
from jax.experimental.pallas import tpu_sc as plsc


def _gather_kernel(n_rows, num_cores, num_workers, seq, nbuf):
  per_w = n_rows // num_workers
  half = 128
  n_chunks = per_w // half
  groups = n_chunks // nbuf

  mesh = plsc.VectorSubcoreMesh(core_axis_name="c", subcore_axis_name="s")

  scratch = (
      [pltpu.VMEM((n_chunks, half), jnp.int32)]
      + [pltpu.VMEM((half, 128), jnp.float32) for _ in range(nbuf)]
      + [pltpu.SemaphoreType.DMA for _ in range(2 * nbuf + 1)]
  )

  @functools.partial(
      pl.kernel,
      out_type=jax.ShapeDtypeStruct((n_rows, 128), jnp.float32),
      mesh=mesh,
      scratch_types=scratch,
      compiler_params=pltpu.CompilerParams(use_tc_tiling_on_sc=True),
  )
  def run(table, idx_hbm, out, idx_v, *rest):
    bufs = rest[:nbuf]
    gsem = rest[nbuf:2 * nbuf]
    osem = rest[2 * nbuf:3 * nbuf]
    isem = rest[3 * nbuf]

    wid = lax.axis_index("s") * num_cores + lax.axis_index("c")
    base = wid * per_w

    pltpu.async_copy(idx_hbm.at[wid], idx_v, isem).wait()

    def start_gather(k, b):
      pltpu.async_copy(table.at[idx_v.at[k]], bufs[b], gsem[b])

    def wait_gather(b):
      pltpu.make_async_copy(out.at[pl.ds(base, half)], bufs[b], gsem[b]).wait()

    def start_out(k, b):
      pltpu.async_copy(bufs[b], out.at[pl.ds(base + k * half, half)], osem[b])

    def wait_out(b):
      pltpu.make_async_copy(bufs[b], out.at[pl.ds(base, half)], osem[b]).wait()

    for b in range(nbuf):
      start_gather(b, b)

    def group_body(g, carry):
      for b in range(nbuf):
        wait_gather(b)
        start_out(g * nbuf + b, b)
      for b in range(nbuf):
        wait_out(b)
        start_gather((g + 1) * nbuf + b, b)
      return carry

    lax.fori_loop(0, groups - 1, group_body, 0)

    for b in range(nbuf):
      wait_gather(b)
      start_out((groups - 1) * nbuf + b, b)
    for b in range(nbuf):
      wait_out(b)

  return run


def kernel(inlets, weight):
  b, s = inlets.shape
  v, d = weight.shape

  n = b * s
  info = plsc.get_sparse_core_info()
  num_workers = info.num_cores * info.num_subcores

  w128 = jnp.pad(weight, ((0, 0), (0, 128 - d)))
  idx = inlets.astype(jnp.int32).reshape(num_workers,
                                         n // (num_workers * 128), 128)
  run = _gather_kernel(n, info.num_cores, num_workers, s, 4)
  out = run(w128, idx)
  return out[:, :d].reshape(b, s, d)


# speedup vs baseline: 1.9293x; 1.0023x over previous
import functools

import jax
import jax.numpy as jnp
from jax import lax
from jax.experimental import pallas as pl
from jax.experimental.pallas import tpu as pltpu
from jax.experimental.pallas import tpu_sc as plsc


def _gather_kernel(n_rows, num_cores, num_workers, nbuf):
  per_w = n_rows // num_workers
  half = 128
  n_chunks = per_w // half
  groups = n_chunks // nbuf

  mesh = plsc.VectorSubcoreMesh(core_axis_name="c", subcore_axis_name="s")

  scratch = (
      [pltpu.VMEM((n_chunks, half), jnp.int32)]
      + [pltpu.VMEM((half, 128), jnp.float32) for _ in range(nbuf)]
      + [pltpu.SemaphoreType.DMA for _ in range(2 * nbuf + 1)]
  )

  @functools.partial(
      pl.kernel,
      out_type=jax.ShapeDtypeStruct((n_rows, 128), jnp.float32),
      mesh=mesh,
      scratch_types=scratch,
      compiler_params=pltpu.CompilerParams(use_tc_tiling_on_sc=True),
  )
  def run(table, idx_hbm, out, idx_v, *rest):
    bufs = rest[:nbuf]
    gsem = rest[nbuf:2 * nbuf]
    osem = rest[2 * nbuf:3 * nbuf]
    isem = rest[3 * nbuf]

    wid = lax.axis_index("s") * num_cores + lax.axis_index("c")
    base = wid * per_w

    pltpu.async_copy(idx_hbm.at[wid], idx_v, isem).wait()

    def start_gather(k, b):
      pltpu.async_copy(table.at[idx_v.at[k]], bufs[b], gsem[b])

    def wait_gather(b):
      pltpu.make_async_copy(out.at[pl.ds(base, half)], bufs[b], gsem[b]).wait()

    def start_out(k, b):
      pltpu.async_copy(bufs[b], out.at[pl.ds(base + k * half, half)], osem[b])

    def wait_out(b):
      pltpu.make_async_copy(bufs[b], out.at[pl.ds(base, half)], osem[b]).wait()

    for b in range(nbuf):
      start_gather(b, b)

    def group_body(g, carry):
      for b in range(nbuf):
        wait_gather(b)
        start_out(g * nbuf + b, b)
      for b in range(nbuf):
        wait_out(b)
        start_gather((g + 1) * nbuf + b, b)
      return carry

    lax.fori_loop(0, groups - 1, group_body, 0)

    for b in range(nbuf):
      wait_gather(b)
      start_out((groups - 1) * nbuf + b, b)
    for b in range(nbuf):
      wait_out(b)

  return run


def kernel(inlets, weight):
  b, s = inlets.shape
  v, d = weight.shape

  n = b * s
  info = plsc.get_sparse_core_info()
  num_workers = info.num_cores * info.num_subcores

  w128 = jnp.pad(weight, ((0, 0), (0, 128 - d)))
  idx = inlets.astype(jnp.int32).reshape(num_workers,
                                         n // (num_workers * 128), 128)
  run = _gather_kernel(n, info.num_cores, num_workers, 4)
  out = run(w128, idx)
  return out[:, :d].reshape(b, s, d)
